# baseline (speedup 1.0000x reference)
import functools

import jax
import jax.numpy as jnp
from jax.experimental import pallas as pl
from jax.experimental.pallas import tpu as pltpu

TOP_K = 8
N_EXPERTS = 64
TILE_N = 1024
HALF_N = TILE_N // 2


def _router_body(x1_ref, x2_ref, wt_ref, out_ref, idx_ref):
    wt = wt_ref[...]
    logits = jnp.concatenate(
        [
            jnp.dot(x1_ref[...], wt, preferred_element_type=jnp.float32),
            jnp.dot(x2_ref[...], wt, preferred_element_type=jnp.float32),
        ],
        axis=0,
    )
    lt = logits.T
    tile_n = lt.shape[1]
    erow = jax.lax.broadcasted_iota(jnp.int32, lt.shape, 0)
    krow = jax.lax.broadcasted_iota(jnp.int32, (TOP_K, tile_n), 0)

    work = lt
    out_t = jnp.zeros_like(lt)
    iacc = jnp.zeros((TOP_K, tile_n), jnp.int32)
    ssum = jnp.zeros((1, tile_n), jnp.float32)
    m0 = None
    for j in range(TOP_K):
        m = jnp.max(work, axis=0, keepdims=True)
        idx = jnp.min(
            jnp.where(work == m, erow, N_EXPERTS), axis=0, keepdims=True
        )
        if j == 0:
            m0 = m
        e = jnp.exp(m - m0)
        ssum = ssum + e
        onehot = erow == idx
        out_t = out_t + jnp.where(onehot, e, 0.0)
        iacc = iacc + jnp.where(krow == j, idx, 0)
        work = jnp.where(onehot, -jnp.inf, work)

    out_ref[...] = (out_t / ssum).T
    idx_ref[...] = iacc.T


@jax.jit
def _router(hidden_states, wt):
    n, d = hidden_states.shape
    e = wt.shape[1]
    grid = (n // TILE_N,)
    return pl.pallas_call(
        _router_body,
        grid=grid,
        in_specs=[
            pl.BlockSpec((HALF_N, d), lambda i: (2 * i, 0)),
            pl.BlockSpec((HALF_N, d), lambda i: (2 * i + 1, 0)),
            pl.BlockSpec((d, e), lambda i: (0, 0)),
        ],
        out_specs=[
            pl.BlockSpec((TILE_N, e), lambda i: (i, 0)),
            pl.BlockSpec((TILE_N, TOP_K), lambda i: (i, 0)),
        ],
        out_shape=[
            jax.ShapeDtypeStruct((n, e), jnp.float32),
            jax.ShapeDtypeStruct((n, TOP_K), jnp.int32),
        ],
        compiler_params=pltpu.CompilerParams(
            dimension_semantics=("parallel",),
        ),
    )(hidden_states, hidden_states, wt)


def kernel(hidden_states, W_route, W_noise):
    del W_noise
    router_output, indices = _router(hidden_states, W_route.T)
    return (router_output, indices)


# speedup vs baseline: 1.0022x; 1.0022x over previous
import functools

import jax
import jax.numpy as jnp
from jax.experimental import pallas as pl
from jax.experimental.pallas import tpu as pltpu

TOP_K = 8
N_EXPERTS = 64
TILE_N = 512
NUM_SLOTS = 4
LOOKAHEAD = 3


def _router_body(x_hbm, wt_ref, out_ref, idx_ref, xbuf, sem):
    i = pl.program_id(0)
    num_steps = pl.num_programs(0)

    def copy_for(step):
        slot = jax.lax.rem(step, NUM_SLOTS)
        return pltpu.make_async_copy(
            x_hbm.at[pl.ds(step * TILE_N, TILE_N), :],
            xbuf.at[slot],
            sem.at[slot],
        )

    @pl.when(i == 0)
    def _prologue():
        for j in range(LOOKAHEAD):
            copy_for(j).start()

    @pl.when(i + LOOKAHEAD < num_steps)
    def _prefetch():
        copy_for(i + LOOKAHEAD).start()

    copy_for(i).wait()

    x = xbuf[jax.lax.rem(i, NUM_SLOTS)]
    wt = wt_ref[...]
    logits = jnp.dot(x, wt, preferred_element_type=jnp.float32)
    lt = logits.T
    tile_n = lt.shape[1]
    erow = jax.lax.broadcasted_iota(jnp.int32, lt.shape, 0)
    krow = jax.lax.broadcasted_iota(jnp.int32, (TOP_K, tile_n), 0)

    work = lt
    out_t = jnp.zeros_like(lt)
    iacc = jnp.zeros((TOP_K, tile_n), jnp.int32)
    ssum = jnp.zeros((1, tile_n), jnp.float32)
    m0 = None
    for j in range(TOP_K):
        m = jnp.max(work, axis=0, keepdims=True)
        idx = jnp.min(
            jnp.where(work == m, erow, N_EXPERTS), axis=0, keepdims=True
        )
        if j == 0:
            m0 = m
        e = jnp.exp(m - m0)
        ssum = ssum + e
        onehot = erow == idx
        out_t = out_t + jnp.where(onehot, e, 0.0)
        iacc = iacc + jnp.where(krow == j, idx, 0)
        work = jnp.where(onehot, -jnp.inf, work)

    out_ref[...] = (out_t / ssum).T
    idx_ref[...] = iacc.T


@jax.jit
def _router(hidden_states, wt):
    n, d = hidden_states.shape
    e = wt.shape[1]
    grid = (n // TILE_N,)
    return pl.pallas_call(
        _router_body,
        grid=grid,
        in_specs=[
            pl.BlockSpec(memory_space=pl.ANY),
            pl.BlockSpec((d, e), lambda i: (0, 0)),
        ],
        out_specs=[
            pl.BlockSpec((TILE_N, e), lambda i: (i, 0)),
            pl.BlockSpec((TILE_N, TOP_K), lambda i: (i, 0)),
        ],
        out_shape=[
            jax.ShapeDtypeStruct((n, e), jnp.float32),
            jax.ShapeDtypeStruct((n, TOP_K), jnp.int32),
        ],
        scratch_shapes=[
            pltpu.VMEM((NUM_SLOTS, TILE_N, d), jnp.float32),
            pltpu.SemaphoreType.DMA((NUM_SLOTS,)),
        ],
        compiler_params=pltpu.CompilerParams(
            dimension_semantics=("arbitrary",),
        ),
    )(hidden_states, wt)


def kernel(hidden_states, W_route, W_noise):
    del W_noise
    router_output, indices = _router(hidden_states, W_route.T)
    return (router_output, indices)
